# Initial kernel scaffold; baseline (speedup 1.0000x reference)
import jax
import jax.numpy as jnp
from jax.experimental import pallas as pl


def kernel(x, edge_index, batch, W1, att_src1, att_dst1, bias1, W2, att_src2, att_dst2, bias2):
    raise NotImplementedError("write your pallas kernel here")


# speedup vs baseline: 16.9432x; 16.9432x over previous
import functools

import jax
import jax.numpy as jnp
from jax import lax
from jax.experimental import pallas as pl
from jax.experimental.pallas import tpu as pltpu
from jax.experimental.pallas import tpu_sc as plsc

N = 10000
E = 320000
F_IN = 128
HID = 64
HEADS = 8
OUT = 64
G = 64

NC = 2
NS = 16
L = 16

TRASH = N
NPAD = 10240
ROWS = NPAD // NS
RB = 1024

EB = 128
E_TOT = E + N
NBLK_ALL = 81
E_PAD = NC * NS * EB * NBLK_ALL
EPT = E_PAD // (NC * NS)
EPS = E_PAD // NS
NBLK_SC = EPS // EB

_MESH = plsc.VectorSubcoreMesh(
    core_axis_name="c", subcore_axis_name="s", num_cores=NC, num_subcores=NS)

f32 = jnp.float32
i32 = jnp.int32


def _dot(a, b):
  return jnp.dot(a, b, precision=lax.Precision.HIGHEST,
                 preferred_element_type=f32)


def _tc1_body(x_ref, w1_ref, ssrc_ref, sdst_ref,
              hc0, hc1, hc2, hc3, asrc16, adst16):
  h = _dot(x_ref[:], w1_ref[:])
  hc0[:] = h[:, 0:128]
  hc1[:] = h[:, 128:256]
  hc2[:] = h[:, 256:384]
  hc3[:] = h[:, 384:512]
  a_s = _dot(h, ssrc_ref[:])
  a_d = _dot(h, sdst_ref[:])
  z8 = jnp.zeros((RB, L - HEADS), f32)
  asrc16[:] = jnp.concatenate([a_s, z8], axis=1)
  adst16[:] = jnp.concatenate([a_d, z8], axis=1)


def _tc2_body(o0, o1, o2, o3, b1_ref, w2_ref, s2s_ref, s2d_ref,
              h2c, a2src16, a2dst16):
  i = pl.program_id(0)
  h1 = jnp.concatenate(
      [o0[:], o1[:], o2[:], o3[:]], axis=1) + b1_ref[:]
  h1 = jnp.where(h1 > 0, h1, jnp.exp(h1) - 1.0)
  rowid = i * RB + lax.broadcasted_iota(i32, (RB, 1), 0)
  h2 = _dot(h1, w2_ref[:])
  h2 = jnp.where(rowid < N, h2, 0.0)
  h2c[:] = h2
  a_s = _dot(h2, s2s_ref[:])
  a_d = _dot(h2, s2d_ref[:])
  z15 = jnp.zeros((RB, L - 1), f32)
  a2src16[:] = jnp.concatenate([a_s, z15], axis=1)
  a2dst16[:] = jnp.concatenate([a_d, z15], axis=1)


def _recip_body(dpart_ref, r_ref):
  r_ref[:] = 1.0 / (dpart_ref[0] + dpart_ref[1] + 1e-16)


def _tc3_body(p0, p1, b2_ref, batch_ref, pooled_ref, h_ref):
  hs = p0[:] + p1[:]
  rowid = lax.broadcasted_iota(i32, (NPAD, 1), 0)
  hz = jnp.where(rowid < N, hs, 0.0)
  mask = (batch_ref[:][None, :] ==
          lax.broadcasted_iota(i32, (G, NPAD), 0)).astype(f32)
  s = _dot(mask, hz)
  cnt = jnp.sum(mask, axis=1, keepdims=True)
  pooled_ref[:] = s / jnp.maximum(cnt, 1.0) + b2_ref[:]
  h_ref[:] = hz[:N, :] + b2_ref[:]


@functools.partial(
    pl.kernel,
    out_type=[jax.ShapeDtypeStruct((E_PAD, L), f32),
              jax.ShapeDtypeStruct((NC, NPAD, L), f32)],
    mesh=_MESH,
    compiler_params=pltpu.CompilerParams(use_tc_tiling_on_sc=False),
    scratch_types=[
        pltpu.VMEM((EB,), i32),
        pltpu.VMEM((EB,), i32),
        pltpu.VMEM((EB, L), f32),
        pltpu.VMEM((EB, L), f32),
        pltpu.VMEM((EB, L), f32),
        pltpu.VMEM_SHARED((NPAD, L), f32),
        pltpu.SemaphoreType.DMA,
    ])
def _edge_coef(asrc_hbm, adst_hbm, src_hbm, dst_hbm, z16_hbm,
               ex_hbm, dpart_hbm,
               srcv, dstv, abuf, bbuf, exbuf, dacc, sem):
  cid = lax.axis_index("c")
  sid = lax.axis_index("s")
  wid = sid * NC + cid
  pltpu.sync_copy(z16_hbm.at[pl.ds(sid * ROWS, ROWS)],
                  dacc.at[pl.ds(sid * ROWS, ROWS)])
  plsc.subcore_barrier()
  base = wid * EPT

  def blk(b, carry):
    eoff = base + b * EB
    pltpu.sync_copy(src_hbm.at[pl.ds(eoff, EB)], srcv)
    pltpu.sync_copy(dst_hbm.at[pl.ds(eoff, EB)], dstv)
    pltpu.async_copy(asrc_hbm.at[srcv], abuf, sem).wait()
    pltpu.async_copy(adst_hbm.at[dstv], bbuf, sem).wait()

    def inner(i, c2):
      v = abuf[i, :] + bbuf[i, :]
      v = jnp.where(v >= 0.0, v, 0.2 * v)
      exbuf[i, :] = jnp.exp(v)
      return c2

    lax.fori_loop(0, EB, inner, 0)
    pltpu.sync_copy(exbuf, ex_hbm.at[pl.ds(eoff, EB)])
    pltpu.sync_copy(exbuf, dacc.at[dstv], add=True)
    return carry

  lax.fori_loop(0, NBLK_ALL, blk, 0)
  plsc.subcore_barrier()
  pltpu.sync_copy(dacc.at[pl.ds(sid * ROWS, ROWS)],
                  dpart_hbm.at[cid, pl.ds(sid * ROWS, ROWS)])


def _lane_splat(v16, lane):
  idx = jnp.full((L, 1), lane, i32)
  return lax.gather(
      v16, idx,
      lax.GatherDimensionNumbers(offset_dims=(), collapsed_slice_dims=(0,),
                                 start_index_map=(0,)),
      slice_sizes=(1,),
      mode=lax.GatherScatterMode.PROMISE_IN_BOUNDS)


def _scale_rows(hbuf, exbuf, rdbuf, i, lanes, nv):
  at16 = exbuf[i, :] * rdbuf[i, :]
  scales = [_lane_splat(at16, l) for l in lanes]
  per = nv // len(lanes)
  for v in range(nv):
    sc = scales[v // per]
    hbuf[i, pl.ds(v * L, L)] = hbuf[i, pl.ds(v * L, L)] * sc


@functools.partial(
    pl.kernel,
    out_type=[jax.ShapeDtypeStruct((NPAD, 128), f32)] * 4,
    mesh=_MESH,
    compiler_params=pltpu.CompilerParams(use_tc_tiling_on_sc=False),
    scratch_types=[
        pltpu.VMEM((EB,), i32),
        pltpu.VMEM((EB,), i32),
        pltpu.VMEM((EB, L), f32),
        pltpu.VMEM((EB, L), f32),
        pltpu.VMEM((EB, 128), f32),
        pltpu.VMEM_SHARED((NPAD, 128), f32),
        pltpu.SemaphoreType.DMA,
    ])
def _message1(hc0, hc1, hc2, hc3, ex_hbm, rd_hbm, src_hbm, dst_hbm, z128_hbm,
              o0, o1, o2, o3,
              srcv, dstv, exbuf, rdbuf, hbuf, acc, sem):
  cid = lax.axis_index("c")
  sid = lax.axis_index("s")

  def do_chunk(hc, out_hbm, lanes):
    pltpu.sync_copy(z128_hbm.at[pl.ds(sid * ROWS, ROWS)],
                    acc.at[pl.ds(sid * ROWS, ROWS)])
    plsc.subcore_barrier()
    base = sid * EPS

    def blk(b, carry):
      eoff = base + b * EB
      pltpu.sync_copy(src_hbm.at[pl.ds(eoff, EB)], srcv)
      pltpu.sync_copy(dst_hbm.at[pl.ds(eoff, EB)], dstv)
      pltpu.async_copy(hc.at[srcv], hbuf, sem).wait()
      pltpu.async_copy(rd_hbm.at[dstv], rdbuf, sem).wait()
      pltpu.sync_copy(ex_hbm.at[pl.ds(eoff, EB)], exbuf)

      def inner(i, c2):
        _scale_rows(hbuf, exbuf, rdbuf, i, lanes, 8)
        return c2

      lax.fori_loop(0, EB, inner, 0)
      pltpu.sync_copy(hbuf, acc.at[dstv], add=True)
      return carry

    lax.fori_loop(0, NBLK_SC, blk, 0)
    plsc.subcore_barrier()
    pltpu.sync_copy(acc.at[pl.ds(sid * ROWS, ROWS)],
                    out_hbm.at[pl.ds(sid * ROWS, ROWS)])
    plsc.subcore_barrier()

  @pl.when(cid == 0)
  def _():
    do_chunk(hc0, o0, (0, 1))
    do_chunk(hc1, o1, (2, 3))

  @pl.when(cid == 1)
  def _():
    do_chunk(hc2, o2, (4, 5))
    do_chunk(hc3, o3, (6, 7))


@functools.partial(
    pl.kernel,
    out_type=[jax.ShapeDtypeStruct((NC, NPAD, OUT), f32)],
    mesh=_MESH,
    compiler_params=pltpu.CompilerParams(use_tc_tiling_on_sc=False),
    scratch_types=[
        pltpu.VMEM((EB,), i32),
        pltpu.VMEM((EB,), i32),
        pltpu.VMEM((EB, L), f32),
        pltpu.VMEM((EB, L), f32),
        pltpu.VMEM((EB, OUT), f32),
        pltpu.VMEM_SHARED((NPAD, OUT), f32),
        pltpu.SemaphoreType.DMA,
    ])
def _message2(h2c, ex_hbm, rd_hbm, src_hbm, dst_hbm, z64_hbm,
              opart,
              srcv, dstv, exbuf, rdbuf, hbuf, acc, sem):
  cid = lax.axis_index("c")
  sid = lax.axis_index("s")
  wid = sid * NC + cid
  pltpu.sync_copy(z64_hbm.at[pl.ds(sid * ROWS, ROWS)],
                  acc.at[pl.ds(sid * ROWS, ROWS)])
  plsc.subcore_barrier()
  base = wid * EPT

  def blk(b, carry):
    eoff = base + b * EB
    pltpu.sync_copy(src_hbm.at[pl.ds(eoff, EB)], srcv)
    pltpu.sync_copy(dst_hbm.at[pl.ds(eoff, EB)], dstv)
    pltpu.async_copy(h2c.at[srcv], hbuf, sem).wait()
    pltpu.async_copy(rd_hbm.at[dstv], rdbuf, sem).wait()
    pltpu.sync_copy(ex_hbm.at[pl.ds(eoff, EB)], exbuf)

    def inner(i, c2):
      _scale_rows(hbuf, exbuf, rdbuf, i, (0,), 4)
      return c2

    lax.fori_loop(0, EB, inner, 0)
    pltpu.sync_copy(hbuf, acc.at[dstv], add=True)
    return carry

  lax.fori_loop(0, NBLK_ALL, blk, 0)
  plsc.subcore_barrier()
  pltpu.sync_copy(acc.at[pl.ds(sid * ROWS, ROWS)],
                  opart.at[cid, pl.ds(sid * ROWS, ROWS)])


def kernel(x, edge_index, batch, W1, att_src1, att_dst1, bias1,
           W2, att_src2, att_dst2, bias2):
  loop = jnp.arange(N, dtype=jnp.int32)
  src = jnp.concatenate(
      [edge_index[0].astype(jnp.int32), loop,
       jnp.zeros((E_PAD - E_TOT,), jnp.int32)])
  dst = jnp.concatenate(
      [edge_index[1].astype(jnp.int32), loop,
       jnp.full((E_PAD - E_TOT,), TRASH, jnp.int32)])
  batch_pad = jnp.concatenate(
      [batch.astype(jnp.int32), jnp.full((NPAD - N,), G, jnp.int32)])

  eye8 = jnp.eye(HEADS, dtype=f32)
  s1src = (att_src1[0][:, :, None] * eye8[:, None, :]).reshape(
      HEADS * HID, HEADS)
  s1dst = (att_dst1[0][:, :, None] * eye8[:, None, :]).reshape(
      HEADS * HID, HEADS)
  s2src = att_src2[0].reshape(OUT, 1)
  s2dst = att_dst2[0].reshape(OUT, 1)

  z128 = jnp.zeros((NPAD, 128), f32)
  z16 = jnp.zeros((NPAD, L), f32)
  z64 = jnp.zeros((NPAD, OUT), f32)

  xp = jnp.concatenate([x, jnp.zeros((NPAD - N, F_IN), f32)], axis=0)

  def _row_blk(cols):
    return pl.BlockSpec((RB, cols), lambda i: (i, 0))

  def _full(shape):
    return pl.BlockSpec(shape, lambda i: tuple(0 for _ in shape))

  hc0, hc1, hc2, hc3, asrc16, adst16 = pl.pallas_call(
      _tc1_body,
      grid=(NPAD // RB,),
      in_specs=[_row_blk(F_IN), _full((F_IN, HEADS * HID)),
                _full((HEADS * HID, HEADS)), _full((HEADS * HID, HEADS))],
      out_specs=[_row_blk(128)] * 4 + [_row_blk(L)] * 2,
      out_shape=[jax.ShapeDtypeStruct((NPAD, 128), f32)] * 4 +
                [jax.ShapeDtypeStruct((NPAD, L), f32)] * 2,
  )(xp, W1, s1src, s1dst)

  ex1, dpart1 = _edge_coef(asrc16, adst16, src, dst, z16)

  rd1 = pl.pallas_call(
      _recip_body,
      out_shape=jax.ShapeDtypeStruct((NPAD, L), f32),
  )(dpart1)

  o0, o1, o2, o3 = _message1(hc0, hc1, hc2, hc3, ex1, rd1, src, dst, z128)

  h2c, a2src16, a2dst16 = pl.pallas_call(
      _tc2_body,
      grid=(NPAD // RB,),
      in_specs=[_row_blk(128)] * 4 +
               [_full((HEADS * HID,)), _full((HEADS * HID, OUT)),
                _full((OUT, 1)), _full((OUT, 1))],
      out_specs=[_row_blk(OUT), _row_blk(L), _row_blk(L)],
      out_shape=[jax.ShapeDtypeStruct((NPAD, OUT), f32),
                 jax.ShapeDtypeStruct((NPAD, L), f32),
                 jax.ShapeDtypeStruct((NPAD, L), f32)],
  )(o0, o1, o2, o3, bias1, W2, s2src, s2dst)

  ex2, dpart2 = _edge_coef(a2src16, a2dst16, src, dst, z16)

  rd2 = pl.pallas_call(
      _recip_body,
      out_shape=jax.ShapeDtypeStruct((NPAD, L), f32),
  )(dpart2)

  (opart,) = _message2(h2c, ex2, rd2, src, dst, z64)

  pooled, h = pl.pallas_call(
      _tc3_body,
      out_shape=[jax.ShapeDtypeStruct((G, OUT), f32),
                 jax.ShapeDtypeStruct((N, OUT), f32)],
  )(opart[0], opart[1], bias2, batch_pad)

  return (pooled, h)


# speedup vs baseline: 20.6936x; 1.2214x over previous
import functools

import jax
import jax.numpy as jnp
from jax import lax
from jax.experimental import pallas as pl
from jax.experimental.pallas import tpu as pltpu
from jax.experimental.pallas import tpu_sc as plsc

N = 10000
E = 320000
F_IN = 128
HID = 64
HEADS = 8
OUT = 64
G = 64

NC = 2
NS = 16
L = 16

TRASH = N
NPAD = 10240
ROWS = NPAD // NS
RB = 1024

EB = 128
E_TOT = E + N
NBLK_ALL = 84
E_PAD = NC * NS * EB * NBLK_ALL
EPT = E_PAD // (NC * NS)
EPS = E_PAD // NS
NBLK_SC = EPS // EB

_MESH = plsc.VectorSubcoreMesh(
    core_axis_name="c", subcore_axis_name="s", num_cores=NC, num_subcores=NS)

f32 = jnp.float32
i32 = jnp.int32


def _dot(a, b):
  return jnp.dot(a, b, precision=lax.Precision.HIGHEST,
                 preferred_element_type=f32)


def _tc1_body(x_ref, w1_ref, ssrc_ref, sdst_ref,
              hc0, hc1, hc2, hc3, asrc16, adst16):
  h = _dot(x_ref[:], w1_ref[:])
  hc0[:] = h[:, 0:128]
  hc1[:] = h[:, 128:256]
  hc2[:] = h[:, 256:384]
  hc3[:] = h[:, 384:512]
  a_s = _dot(h, ssrc_ref[:])
  a_d = _dot(h, sdst_ref[:])
  z8 = jnp.zeros((RB, L - HEADS), f32)
  asrc16[:] = jnp.concatenate([a_s, z8], axis=1)
  adst16[:] = jnp.concatenate([a_d, z8], axis=1)


def _tc2_body(o0, o1, o2, o3, b1_ref, w2_ref, s2s_ref, s2d_ref,
              h2c, a2src16, a2dst16):
  i = pl.program_id(0)
  h1 = jnp.concatenate(
      [o0[:], o1[:], o2[:], o3[:]], axis=1) + b1_ref[:]
  h1 = jnp.where(h1 > 0, h1, jnp.exp(h1) - 1.0)
  rowid = i * RB + lax.broadcasted_iota(i32, (RB, 1), 0)
  h2 = _dot(h1, w2_ref[:])
  h2 = jnp.where(rowid < N, h2, 0.0)
  h2c[:] = h2
  a_s = _dot(h2, s2s_ref[:])
  a_d = _dot(h2, s2d_ref[:])
  z15 = jnp.zeros((RB, L - 1), f32)
  a2src16[:] = jnp.concatenate([a_s, z15], axis=1)
  a2dst16[:] = jnp.concatenate([a_d, z15], axis=1)


def _recip_body(dpart_ref, r_ref):
  r_ref[:] = 1.0 / (dpart_ref[0] + dpart_ref[1] + 1e-16)


def _tc3_body(p0, p1, b2_ref, batch_ref, pooled_ref, h_ref):
  hs = p0[:] + p1[:]
  rowid = lax.broadcasted_iota(i32, (NPAD, 1), 0)
  hz = jnp.where(rowid < N, hs, 0.0)
  mask = (batch_ref[:][None, :] ==
          lax.broadcasted_iota(i32, (G, NPAD), 0)).astype(f32)
  s = _dot(mask, hz)
  cnt = jnp.sum(mask, axis=1, keepdims=True)
  pooled_ref[:] = s / jnp.maximum(cnt, 1.0) + b2_ref[:]
  h_ref[:] = hz[:N, :] + b2_ref[:]


def _lane_splat(v16, lane):
  idx = jnp.full((L, 1), lane, i32)
  return lax.gather(
      v16, idx,
      lax.GatherDimensionNumbers(offset_dims=(), collapsed_slice_dims=(0,),
                                 start_index_map=(0,)),
      slice_sizes=(1,),
      mode=lax.GatherScatterMode.PROMISE_IN_BOUNDS)


def _scale_rows(hbuf, exbuf, rdbuf, i, lanes, nv):
  at16 = exbuf[i, :] * rdbuf[i, :]
  scales = [_lane_splat(at16, l) for l in lanes]
  per = nv // len(lanes)
  for v in range(nv):
    sc = scales[v // per]
    hbuf[i, pl.ds(v * L, L)] = hbuf[i, pl.ds(v * L, L)] * sc


def _run_pipeline(nblk, prefetch, launch, finish):
  for t in range(4):
    prefetch(t, t)
  launch(0, 0, 0)

  def quad(q, carry):
    b4 = 4 * q
    for c in range(4):
      b = b4 + c
      sn, tn = (c + 1) % 2, (c + 1) % 4

      @pl.when(b + 1 < nblk)
      def _():
        launch(sn, tn, b + 1)

      finish(c % 2, c, b)

      @pl.when(b + 4 < nblk)
      def _():
        prefetch(c, b + 4)
    return carry

  lax.fori_loop(0, nblk // 4, quad, 0)


@functools.partial(
    pl.kernel,
    out_type=[jax.ShapeDtypeStruct((E_PAD, L), f32),
              jax.ShapeDtypeStruct((NC, NPAD, L), f32)],
    mesh=_MESH,
    compiler_params=pltpu.CompilerParams(use_tc_tiling_on_sc=False),
    scratch_types=[
        [pltpu.VMEM((EB,), i32)] * 4,
        [pltpu.VMEM((EB,), i32)] * 4,
        [pltpu.VMEM((EB, L), f32)] * 2,
        [pltpu.VMEM((EB, L), f32)] * 2,
        [pltpu.VMEM((EB, L), f32)] * 2,
        pltpu.VMEM_SHARED((NPAD, L), f32),
        [pltpu.SemaphoreType.DMA] * 4,
        [pltpu.SemaphoreType.DMA] * 2,
    ])
def _edge_coef(asrc_hbm, adst_hbm, src_hbm, dst_hbm, z16_hbm,
               ex_hbm, dpart_hbm,
               srcv, dstv, abuf, bbuf, exbuf, dacc, semi, semg):
  cid = lax.axis_index("c")
  sid = lax.axis_index("s")
  wid = sid * NC + cid
  pltpu.sync_copy(z16_hbm.at[pl.ds(sid * ROWS, ROWS)],
                  dacc.at[pl.ds(sid * ROWS, ROWS)])
  plsc.subcore_barrier()
  ebase = wid * EPT

  def prefetch(t, b):
    off = ebase + b * EB
    pltpu.async_copy(src_hbm.at[pl.ds(off, EB)], srcv[t], semi[t])
    pltpu.async_copy(dst_hbm.at[pl.ds(off, EB)], dstv[t], semi[t])

  def launch(s, t, b):
    pltpu.make_async_copy(src_hbm.at[pl.ds(0, EB)], srcv[t], semi[t]).wait()
    pltpu.make_async_copy(dst_hbm.at[pl.ds(0, EB)], dstv[t], semi[t]).wait()
    pltpu.async_copy(asrc_hbm.at[srcv[t]], abuf[s], semg[s])
    pltpu.async_copy(adst_hbm.at[dstv[t]], bbuf[s], semg[s])

  def finish(s, t, b):
    pltpu.make_async_copy(asrc_hbm.at[srcv[t]], abuf[s], semg[s]).wait()
    pltpu.make_async_copy(adst_hbm.at[dstv[t]], bbuf[s], semg[s]).wait()

    @plsc.parallel_loop(0, EB, unroll=4)
    def body(i):
      v = abuf[s][i, :] + bbuf[s][i, :]
      v = jnp.where(v >= 0.0, v, 0.2 * v)
      exbuf[s][i, :] = jnp.exp(v)

    pltpu.sync_copy(exbuf[s], ex_hbm.at[pl.ds(ebase + b * EB, EB)])
    pltpu.sync_copy(exbuf[s], dacc.at[dstv[t]], add=True)

  _run_pipeline(NBLK_ALL, prefetch, launch, finish)
  plsc.subcore_barrier()
  pltpu.sync_copy(dacc.at[pl.ds(sid * ROWS, ROWS)],
                  dpart_hbm.at[cid, pl.ds(sid * ROWS, ROWS)])


@functools.partial(
    pl.kernel,
    out_type=[jax.ShapeDtypeStruct((NPAD, 128), f32)] * 4,
    mesh=_MESH,
    compiler_params=pltpu.CompilerParams(use_tc_tiling_on_sc=False),
    scratch_types=[
        [pltpu.VMEM((EB,), i32)] * 4,
        [pltpu.VMEM((EB,), i32)] * 4,
        [pltpu.VMEM((EB, L), f32)] * 2,
        [pltpu.VMEM((EB, L), f32)] * 2,
        [pltpu.VMEM((EB, 128), f32)] * 2,
        pltpu.VMEM_SHARED((NPAD, 128), f32),
        [pltpu.SemaphoreType.DMA] * 4,
        [pltpu.SemaphoreType.DMA] * 2,
    ])
def _message1(hc0, hc1, hc2, hc3, ex_hbm, rd_hbm, src_hbm, dst_hbm, z128_hbm,
              o0, o1, o2, o3,
              srcv, dstv, exbuf, rdbuf, hbuf, acc, semi, semg):
  cid = lax.axis_index("c")
  sid = lax.axis_index("s")
  ebase = sid * EPS

  def prefetch(t, b):
    off = ebase + b * EB
    pltpu.async_copy(src_hbm.at[pl.ds(off, EB)], srcv[t], semi[t])
    pltpu.async_copy(dst_hbm.at[pl.ds(off, EB)], dstv[t], semi[t])

  def do_chunk(hc, out_hbm, lanes):
    pltpu.sync_copy(z128_hbm.at[pl.ds(sid * ROWS, ROWS)],
                    acc.at[pl.ds(sid * ROWS, ROWS)])
    plsc.subcore_barrier()

    def launch(s, t, b):
      pltpu.make_async_copy(src_hbm.at[pl.ds(0, EB)], srcv[t], semi[t]).wait()
      pltpu.make_async_copy(dst_hbm.at[pl.ds(0, EB)], dstv[t], semi[t]).wait()
      pltpu.async_copy(hc.at[srcv[t]], hbuf[s], semg[s])
      pltpu.async_copy(rd_hbm.at[dstv[t]], rdbuf[s], semg[s])
      pltpu.async_copy(ex_hbm.at[pl.ds(ebase + b * EB, EB)], exbuf[s], semg[s])

    def finish(s, t, b):
      pltpu.make_async_copy(hc.at[srcv[t]], hbuf[s], semg[s]).wait()
      pltpu.make_async_copy(rd_hbm.at[dstv[t]], rdbuf[s], semg[s]).wait()
      pltpu.make_async_copy(ex_hbm.at[pl.ds(0, EB)], exbuf[s], semg[s]).wait()

      @plsc.parallel_loop(0, EB, unroll=2)
      def body(i):
        _scale_rows(hbuf[s], exbuf[s], rdbuf[s], i, lanes, 8)

      pltpu.sync_copy(hbuf[s], acc.at[dstv[t]], add=True)

    _run_pipeline(NBLK_SC, prefetch, launch, finish)
    plsc.subcore_barrier()
    pltpu.sync_copy(acc.at[pl.ds(sid * ROWS, ROWS)],
                    out_hbm.at[pl.ds(sid * ROWS, ROWS)])
    plsc.subcore_barrier()

  @pl.when(cid == 0)
  def _():
    do_chunk(hc0, o0, (0, 1))
    do_chunk(hc1, o1, (2, 3))

  @pl.when(cid == 1)
  def _():
    do_chunk(hc2, o2, (4, 5))
    do_chunk(hc3, o3, (6, 7))


@functools.partial(
    pl.kernel,
    out_type=[jax.ShapeDtypeStruct((NC, NPAD, OUT), f32)],
    mesh=_MESH,
    compiler_params=pltpu.CompilerParams(use_tc_tiling_on_sc=False),
    scratch_types=[
        [pltpu.VMEM((EB,), i32)] * 4,
        [pltpu.VMEM((EB,), i32)] * 4,
        [pltpu.VMEM((EB, L), f32)] * 2,
        [pltpu.VMEM((EB, L), f32)] * 2,
        [pltpu.VMEM((EB, OUT), f32)] * 2,
        pltpu.VMEM_SHARED((NPAD, OUT), f32),
        [pltpu.SemaphoreType.DMA] * 4,
        [pltpu.SemaphoreType.DMA] * 2,
    ])
def _message2(h2c, ex_hbm, rd_hbm, src_hbm, dst_hbm, z64_hbm,
              opart,
              srcv, dstv, exbuf, rdbuf, hbuf, acc, semi, semg):
  cid = lax.axis_index("c")
  sid = lax.axis_index("s")
  wid = sid * NC + cid
  ebase = wid * EPT
  pltpu.sync_copy(z64_hbm.at[pl.ds(sid * ROWS, ROWS)],
                  acc.at[pl.ds(sid * ROWS, ROWS)])
  plsc.subcore_barrier()

  def prefetch(t, b):
    off = ebase + b * EB
    pltpu.async_copy(src_hbm.at[pl.ds(off, EB)], srcv[t], semi[t])
    pltpu.async_copy(dst_hbm.at[pl.ds(off, EB)], dstv[t], semi[t])

  def launch(s, t, b):
    pltpu.make_async_copy(src_hbm.at[pl.ds(0, EB)], srcv[t], semi[t]).wait()
    pltpu.make_async_copy(dst_hbm.at[pl.ds(0, EB)], dstv[t], semi[t]).wait()
    pltpu.async_copy(h2c.at[srcv[t]], hbuf[s], semg[s])
    pltpu.async_copy(rd_hbm.at[dstv[t]], rdbuf[s], semg[s])
    pltpu.async_copy(ex_hbm.at[pl.ds(ebase + b * EB, EB)], exbuf[s], semg[s])

  def finish(s, t, b):
    pltpu.make_async_copy(h2c.at[srcv[t]], hbuf[s], semg[s]).wait()
    pltpu.make_async_copy(rd_hbm.at[dstv[t]], rdbuf[s], semg[s]).wait()
    pltpu.make_async_copy(ex_hbm.at[pl.ds(0, EB)], exbuf[s], semg[s]).wait()

    @plsc.parallel_loop(0, EB, unroll=2)
    def body(i):
      _scale_rows(hbuf[s], exbuf[s], rdbuf[s], i, (0,), 4)

    pltpu.sync_copy(hbuf[s], acc.at[dstv[t]], add=True)

  _run_pipeline(NBLK_ALL, prefetch, launch, finish)
  plsc.subcore_barrier()
  pltpu.sync_copy(acc.at[pl.ds(sid * ROWS, ROWS)],
                  opart.at[cid, pl.ds(sid * ROWS, ROWS)])


def kernel(x, edge_index, batch, W1, att_src1, att_dst1, bias1,
           W2, att_src2, att_dst2, bias2):
  loop = jnp.arange(N, dtype=jnp.int32)
  src = jnp.concatenate(
      [edge_index[0].astype(jnp.int32), loop,
       jnp.zeros((E_PAD - E_TOT,), jnp.int32)])
  dst = jnp.concatenate(
      [edge_index[1].astype(jnp.int32), loop,
       jnp.full((E_PAD - E_TOT,), TRASH, jnp.int32)])
  batch_pad = jnp.concatenate(
      [batch.astype(jnp.int32), jnp.full((NPAD - N,), G, jnp.int32)])

  eye8 = jnp.eye(HEADS, dtype=f32)
  s1src = (att_src1[0][:, :, None] * eye8[:, None, :]).reshape(
      HEADS * HID, HEADS)
  s1dst = (att_dst1[0][:, :, None] * eye8[:, None, :]).reshape(
      HEADS * HID, HEADS)
  s2src = att_src2[0].reshape(OUT, 1)
  s2dst = att_dst2[0].reshape(OUT, 1)

  z128 = jnp.zeros((NPAD, 128), f32)
  z16 = jnp.zeros((NPAD, L), f32)
  z64 = jnp.zeros((NPAD, OUT), f32)

  xp = jnp.concatenate([x, jnp.zeros((NPAD - N, F_IN), f32)], axis=0)

  def _row_blk(cols):
    return pl.BlockSpec((RB, cols), lambda i: (i, 0))

  def _full(shape):
    return pl.BlockSpec(shape, lambda i: tuple(0 for _ in shape))

  hc0, hc1, hc2, hc3, asrc16, adst16 = pl.pallas_call(
      _tc1_body,
      grid=(NPAD // RB,),
      in_specs=[_row_blk(F_IN), _full((F_IN, HEADS * HID)),
                _full((HEADS * HID, HEADS)), _full((HEADS * HID, HEADS))],
      out_specs=[_row_blk(128)] * 4 + [_row_blk(L)] * 2,
      out_shape=[jax.ShapeDtypeStruct((NPAD, 128), f32)] * 4 +
                [jax.ShapeDtypeStruct((NPAD, L), f32)] * 2,
  )(xp, W1, s1src, s1dst)

  ex1, dpart1 = _edge_coef(asrc16, adst16, src, dst, z16)

  rd1 = pl.pallas_call(
      _recip_body,
      out_shape=jax.ShapeDtypeStruct((NPAD, L), f32),
  )(dpart1)

  o0, o1, o2, o3 = _message1(hc0, hc1, hc2, hc3, ex1, rd1, src, dst, z128)

  h2c, a2src16, a2dst16 = pl.pallas_call(
      _tc2_body,
      grid=(NPAD // RB,),
      in_specs=[_row_blk(128)] * 4 +
               [_full((HEADS * HID,)), _full((HEADS * HID, OUT)),
                _full((OUT, 1)), _full((OUT, 1))],
      out_specs=[_row_blk(OUT), _row_blk(L), _row_blk(L)],
      out_shape=[jax.ShapeDtypeStruct((NPAD, OUT), f32),
                 jax.ShapeDtypeStruct((NPAD, L), f32),
                 jax.ShapeDtypeStruct((NPAD, L), f32)],
  )(o0, o1, o2, o3, bias1, W2, s2src, s2dst)

  ex2, dpart2 = _edge_coef(a2src16, a2dst16, src, dst, z16)

  rd2 = pl.pallas_call(
      _recip_body,
      out_shape=jax.ShapeDtypeStruct((NPAD, L), f32),
  )(dpart2)

  (opart,) = _message2(h2c, ex2, rd2, src, dst, z64)

  pooled, h = pl.pallas_call(
      _tc3_body,
      out_shape=[jax.ShapeDtypeStruct((G, OUT), f32),
                 jax.ShapeDtypeStruct((N, OUT), f32)],
  )(opart[0], opart[1], bias2, batch_pad)

  return (pooled, h)


# speedup vs baseline: 26.3945x; 1.2755x over previous
import functools

import jax
import jax.numpy as jnp
from jax import lax
from jax.experimental import pallas as pl
from jax.experimental.pallas import tpu as pltpu
from jax.experimental.pallas import tpu_sc as plsc

N = 10000
E = 320000
F_IN = 128
HID = 64
HEADS = 8
OUT = 64
G = 64

NC = 2
NS = 16
L = 16

TRASH = N
NPAD = 10240
ROWS = NPAD // NS
RB = 1024

EB = 128
E_TOT = E + N
NBLK_ALL = 84
E_PAD = NC * NS * EB * NBLK_ALL
EPT = E_PAD // (NC * NS)
EPS = E_PAD // NS
NBLK_SC = EPS // EB
EBM = 64
NBLK_M = EPS // EBM

_MESH = plsc.VectorSubcoreMesh(
    core_axis_name="c", subcore_axis_name="s", num_cores=NC, num_subcores=NS)

f32 = jnp.float32
i32 = jnp.int32


def _dot(a, b):
  return jnp.dot(a, b, precision=lax.Precision.HIGHEST,
                 preferred_element_type=f32)


def _tc1_body(x_ref, w1_ref, ssrc_ref, sdst_ref,
              hc0, hc1, hc2, hc3, asrc16, adst16):
  h = _dot(x_ref[:], w1_ref[:])
  hb = h.astype(jnp.bfloat16)
  hc0[:] = hb[:, 0:128]
  hc1[:] = hb[:, 128:256]
  hc2[:] = hb[:, 256:384]
  hc3[:] = hb[:, 384:512]
  a_s = _dot(h, ssrc_ref[:])
  a_d = _dot(h, sdst_ref[:])
  z8 = jnp.zeros((RB, L - HEADS), f32)
  asrc16[:] = jnp.concatenate([a_s, z8], axis=1)
  adst16[:] = jnp.concatenate([a_d, z8], axis=1)


def _tc2_body(o0, o1, o2, o3, b1_ref, w2_ref, s2s_ref, s2d_ref,
              h2c, a2src16, a2dst16):
  i = pl.program_id(0)
  h1 = jnp.concatenate(
      [o0[:], o1[:], o2[:], o3[:]], axis=1) + b1_ref[:]
  h1 = jnp.where(h1 > 0, h1, jnp.exp(h1) - 1.0)
  rowid = i * RB + lax.broadcasted_iota(i32, (RB, 1), 0)
  h2 = _dot(h1, w2_ref[:])
  h2 = jnp.where(rowid < N, h2, 0.0)
  h2c[:] = h2
  a_s = _dot(h2, s2s_ref[:])
  a_d = _dot(h2, s2d_ref[:])
  z15 = jnp.zeros((RB, L - 1), f32)
  a2src16[:] = jnp.concatenate([a_s, z15], axis=1)
  a2dst16[:] = jnp.concatenate([a_d, z15], axis=1)


def _recip_body(dpart_ref, r_ref):
  r_ref[:] = 1.0 / (dpart_ref[0] + dpart_ref[1] + 1e-16)


def _tc3_body(p0, p1, b2_ref, batch_ref, pooled_ref, h_ref):
  hs = p0[:] + p1[:]
  rowid = lax.broadcasted_iota(i32, (NPAD, 1), 0)
  hz = jnp.where(rowid < N, hs, 0.0)
  mask = (batch_ref[:][None, :] ==
          lax.broadcasted_iota(i32, (G, NPAD), 0)).astype(f32)
  s = _dot(mask, hz)
  cnt = jnp.sum(mask, axis=1, keepdims=True)
  pooled_ref[:] = s / jnp.maximum(cnt, 1.0) + b2_ref[:]
  h_ref[:] = hz[:N, :] + b2_ref[:]


def _lane_splat(v16, lane):
  idx = jnp.full((L, 1), lane, i32)
  return lax.gather(
      v16, idx,
      lax.GatherDimensionNumbers(offset_dims=(), collapsed_slice_dims=(0,),
                                 start_index_map=(0,)),
      slice_sizes=(1,),
      mode=lax.GatherScatterMode.PROMISE_IN_BOUNDS)


def _scale_rows(hbuf, exbuf, rdbuf, i, lanes, nv):
  at16 = exbuf[i, :] * rdbuf[i, :]
  scales = [_lane_splat(at16, l) for l in lanes]
  per = nv // len(lanes)
  for v in range(nv):
    sc = scales[v // per]
    hbuf[i, pl.ds(v * L, L)] = hbuf[i, pl.ds(v * L, L)] * sc


def _scale_rows_packed(hbuf, sbuf, exbuf, rdbuf, i, lanes):
  at16 = exbuf[i, :] * rdbuf[i, :]
  scales = [_lane_splat(at16, l) for l in lanes]
  for g in range(4):
    w = hbuf[i, pl.ds(g * L, L)]
    lo = lax.bitcast_convert_type(lax.shift_left(w, 16), f32)
    hi = lax.bitcast_convert_type(
        lax.bitwise_and(w, jnp.int32(-65536)), f32)
    sc = scales[g // 2]
    sbuf[i, pl.ds(g * 2 * L, L)] = lo * sc
    sbuf[i, pl.ds((g * 2 + 1) * L, L)] = hi * sc


def _run_pipeline(nblk, prefetch, launch, finish):
  for t in range(4):
    prefetch(t, t)
  launch(0, 0, 0)

  def quad(q, carry):
    b4 = 4 * q
    for c in range(4):
      b = b4 + c
      sn, tn = (c + 1) % 2, (c + 1) % 4

      @pl.when(b + 1 < nblk)
      def _():
        launch(sn, tn, b + 1)

      finish(c % 2, c, b)
    return carry

  lax.fori_loop(0, nblk // 4, quad, 0)


@functools.partial(
    pl.kernel,
    out_type=[jax.ShapeDtypeStruct((E_PAD, L), f32),
              jax.ShapeDtypeStruct((NC, NPAD, L), f32)],
    mesh=_MESH,
    compiler_params=pltpu.CompilerParams(use_tc_tiling_on_sc=False),
    scratch_types=[
        [pltpu.VMEM((EB,), i32)] * 4,
        [pltpu.VMEM((EB,), i32)] * 4,
        [pltpu.VMEM((EB, L), f32)] * 2,
        [pltpu.VMEM((EB, L), f32)] * 2,
        [pltpu.VMEM((EB, L), f32)] * 2,
        pltpu.VMEM_SHARED((NPAD, L), f32),
        [pltpu.SemaphoreType.DMA] * 4,
        [pltpu.SemaphoreType.DMA] * 2,
        [pltpu.SemaphoreType.DMA] * 2,
    ])
def _edge_coef(asrc_hbm, adst_hbm, src_hbm, dst_hbm, z16_hbm,
               ex_hbm, dpart_hbm,
               srcv, dstv, abuf, bbuf, exbuf, dacc, semi, semg, semsc):
  cid = lax.axis_index("c")
  sid = lax.axis_index("s")
  wid = sid * NC + cid
  pltpu.sync_copy(z16_hbm.at[pl.ds(sid * ROWS, ROWS)],
                  dacc.at[pl.ds(sid * ROWS, ROWS)])
  plsc.subcore_barrier()
  ebase = wid * EPT

  def prefetch(t, b):
    off = ebase + b * EB
    pltpu.async_copy(src_hbm.at[pl.ds(off, EB)], srcv[t], semi[t])
    pltpu.async_copy(dst_hbm.at[pl.ds(off, EB)], dstv[t], semi[t])

  def launch(s, t, b):
    @pl.when(b >= 2)
    def _():
      pltpu.make_async_copy(exbuf[s], dacc.at[dstv[(t + 2) % 4]], semsc[s]).wait()

      @pl.when(b + 2 < NBLK_ALL)
      def _():
        prefetch((t + 2) % 4, b + 2)

    pltpu.make_async_copy(src_hbm.at[pl.ds(0, EB)], srcv[t], semi[t]).wait()
    pltpu.make_async_copy(dst_hbm.at[pl.ds(0, EB)], dstv[t], semi[t]).wait()
    pltpu.async_copy(asrc_hbm.at[srcv[t]], abuf[s], semg[s])
    pltpu.async_copy(adst_hbm.at[dstv[t]], bbuf[s], semg[s])

  def finish(s, t, b):
    pltpu.make_async_copy(asrc_hbm.at[srcv[t]], abuf[s], semg[s]).wait()
    pltpu.make_async_copy(adst_hbm.at[dstv[t]], bbuf[s], semg[s]).wait()

    @plsc.parallel_loop(0, EB, unroll=4)
    def body(i):
      v = abuf[s][i, :] + bbuf[s][i, :]
      v = jnp.where(v >= 0.0, v, 0.2 * v)
      exbuf[s][i, :] = jnp.exp(v)

    pltpu.sync_copy(exbuf[s], ex_hbm.at[pl.ds(ebase + b * EB, EB)])
    pltpu.async_copy(exbuf[s], dacc.at[dstv[t]], semsc[s], add=True)

  _run_pipeline(NBLK_ALL, prefetch, launch, finish)
  pltpu.make_async_copy(exbuf[0], dacc.at[dstv[2]], semsc[0]).wait()
  pltpu.make_async_copy(exbuf[1], dacc.at[dstv[3]], semsc[1]).wait()
  plsc.subcore_barrier()
  pltpu.sync_copy(dacc.at[pl.ds(sid * ROWS, ROWS)],
                  dpart_hbm.at[cid, pl.ds(sid * ROWS, ROWS)])


@functools.partial(
    pl.kernel,
    out_type=[jax.ShapeDtypeStruct((NPAD, 128), f32)] * 4,
    mesh=_MESH,
    compiler_params=pltpu.CompilerParams(use_tc_tiling_on_sc=False),
    scratch_types=[
        [pltpu.VMEM((EBM,), i32)] * 4,
        [pltpu.VMEM((EBM,), i32)] * 4,
        [pltpu.VMEM((EBM, L), f32)] * 2,
        [pltpu.VMEM((EBM, L), f32)] * 2,
        [pltpu.VMEM((EBM, 64), i32)] * 2,
        [pltpu.VMEM((EBM, 128), f32)] * 2,
        pltpu.VMEM_SHARED((NPAD, 128), f32),
        [pltpu.SemaphoreType.DMA] * 4,
        [pltpu.SemaphoreType.DMA] * 2,
        [pltpu.SemaphoreType.DMA] * 2,
    ])
def _message1(hc0, hc1, hc2, hc3, ex_hbm, rd_hbm, src_hbm, dst_hbm, z128_hbm,
              o0, o1, o2, o3,
              srcv, dstv, exbuf, rdbuf, hbuf, sbuf, acc, semi, semg, semsc):
  cid = lax.axis_index("c")
  sid = lax.axis_index("s")
  ebase = sid * EPS

  def prefetch(t, b):
    off = ebase + b * EBM
    pltpu.async_copy(src_hbm.at[pl.ds(off, EBM)], srcv[t], semi[t])
    pltpu.async_copy(dst_hbm.at[pl.ds(off, EBM)], dstv[t], semi[t])

  def do_chunk(hc, out_hbm, lanes):
    pltpu.sync_copy(z128_hbm.at[pl.ds(sid * ROWS, ROWS)],
                    acc.at[pl.ds(sid * ROWS, ROWS)])
    plsc.subcore_barrier()

    def launch(s, t, b):
      @pl.when(b >= 2)
      def _():
        pltpu.make_async_copy(sbuf[s], acc.at[dstv[(t + 2) % 4]], semsc[s]).wait()

        @pl.when(b + 2 < NBLK_M)
        def _():
          prefetch((t + 2) % 4, b + 2)

      pltpu.make_async_copy(src_hbm.at[pl.ds(0, EBM)], srcv[t], semi[t]).wait()
      pltpu.make_async_copy(dst_hbm.at[pl.ds(0, EBM)], dstv[t], semi[t]).wait()
      pltpu.async_copy(hc.at[srcv[t]], hbuf[s], semg[s])
      pltpu.async_copy(rd_hbm.at[dstv[t]], rdbuf[s], semg[s])
      pltpu.async_copy(ex_hbm.at[pl.ds(ebase + b * EBM, EBM)], exbuf[s], semg[s])

    def finish(s, t, b):
      pltpu.make_async_copy(hc.at[srcv[t]], hbuf[s], semg[s]).wait()
      pltpu.make_async_copy(rd_hbm.at[dstv[t]], rdbuf[s], semg[s]).wait()
      pltpu.make_async_copy(ex_hbm.at[pl.ds(0, EBM)], exbuf[s], semg[s]).wait()

      @plsc.parallel_loop(0, EBM, unroll=2)
      def body(i):
        _scale_rows_packed(hbuf[s], sbuf[s], exbuf[s], rdbuf[s], i, lanes)

      pltpu.async_copy(sbuf[s], acc.at[dstv[t]], semsc[s], add=True)

    _run_pipeline(NBLK_M, prefetch, launch, finish)
    pltpu.make_async_copy(sbuf[0], acc.at[dstv[2]], semsc[0]).wait()
    pltpu.make_async_copy(sbuf[1], acc.at[dstv[3]], semsc[1]).wait()
    plsc.subcore_barrier()
    pltpu.sync_copy(acc.at[pl.ds(sid * ROWS, ROWS)],
                    out_hbm.at[pl.ds(sid * ROWS, ROWS)])
    plsc.subcore_barrier()

  @pl.when(cid == 0)
  def _():
    do_chunk(hc0, o0, (0, 1))
    do_chunk(hc1, o1, (2, 3))

  @pl.when(cid == 1)
  def _():
    do_chunk(hc2, o2, (4, 5))
    do_chunk(hc3, o3, (6, 7))


@functools.partial(
    pl.kernel,
    out_type=[jax.ShapeDtypeStruct((NC, NPAD, OUT), f32)],
    mesh=_MESH,
    compiler_params=pltpu.CompilerParams(use_tc_tiling_on_sc=False),
    scratch_types=[
        [pltpu.VMEM((EB,), i32)] * 4,
        [pltpu.VMEM((EB,), i32)] * 4,
        [pltpu.VMEM((EB, L), f32)] * 2,
        [pltpu.VMEM((EB, L), f32)] * 2,
        [pltpu.VMEM((EB, OUT), f32)] * 2,
        pltpu.VMEM_SHARED((NPAD, OUT), f32),
        [pltpu.SemaphoreType.DMA] * 4,
        [pltpu.SemaphoreType.DMA] * 2,
        [pltpu.SemaphoreType.DMA] * 2,
    ])
def _message2(h2c, ex_hbm, rd_hbm, src_hbm, dst_hbm, z64_hbm,
              opart,
              srcv, dstv, exbuf, rdbuf, hbuf, acc, semi, semg, semsc):
  cid = lax.axis_index("c")
  sid = lax.axis_index("s")
  wid = sid * NC + cid
  ebase = wid * EPT
  pltpu.sync_copy(z64_hbm.at[pl.ds(sid * ROWS, ROWS)],
                  acc.at[pl.ds(sid * ROWS, ROWS)])
  plsc.subcore_barrier()

  def prefetch(t, b):
    off = ebase + b * EB
    pltpu.async_copy(src_hbm.at[pl.ds(off, EB)], srcv[t], semi[t])
    pltpu.async_copy(dst_hbm.at[pl.ds(off, EB)], dstv[t], semi[t])

  def launch(s, t, b):
    @pl.when(b >= 2)
    def _():
      pltpu.make_async_copy(hbuf[s], acc.at[dstv[(t + 2) % 4]], semsc[s]).wait()

      @pl.when(b + 2 < NBLK_ALL)
      def _():
        prefetch((t + 2) % 4, b + 2)

    pltpu.make_async_copy(src_hbm.at[pl.ds(0, EB)], srcv[t], semi[t]).wait()
    pltpu.make_async_copy(dst_hbm.at[pl.ds(0, EB)], dstv[t], semi[t]).wait()
    pltpu.async_copy(h2c.at[srcv[t]], hbuf[s], semg[s])
    pltpu.async_copy(rd_hbm.at[dstv[t]], rdbuf[s], semg[s])
    pltpu.async_copy(ex_hbm.at[pl.ds(ebase + b * EB, EB)], exbuf[s], semg[s])

  def finish(s, t, b):
    pltpu.make_async_copy(h2c.at[srcv[t]], hbuf[s], semg[s]).wait()
    pltpu.make_async_copy(rd_hbm.at[dstv[t]], rdbuf[s], semg[s]).wait()
    pltpu.make_async_copy(ex_hbm.at[pl.ds(0, EB)], exbuf[s], semg[s]).wait()

    @plsc.parallel_loop(0, EB, unroll=2)
    def body(i):
      _scale_rows(hbuf[s], exbuf[s], rdbuf[s], i, (0,), 4)

    pltpu.async_copy(hbuf[s], acc.at[dstv[t]], semsc[s], add=True)

  _run_pipeline(NBLK_ALL, prefetch, launch, finish)
  pltpu.make_async_copy(hbuf[0], acc.at[dstv[2]], semsc[0]).wait()
  pltpu.make_async_copy(hbuf[1], acc.at[dstv[3]], semsc[1]).wait()
  plsc.subcore_barrier()
  pltpu.sync_copy(acc.at[pl.ds(sid * ROWS, ROWS)],
                  opart.at[cid, pl.ds(sid * ROWS, ROWS)])


def kernel(x, edge_index, batch, W1, att_src1, att_dst1, bias1,
           W2, att_src2, att_dst2, bias2):
  loop = jnp.arange(N, dtype=jnp.int32)
  src = jnp.concatenate(
      [edge_index[0].astype(jnp.int32), loop,
       jnp.zeros((E_PAD - E_TOT,), jnp.int32)])
  dst = jnp.concatenate(
      [edge_index[1].astype(jnp.int32), loop,
       jnp.full((E_PAD - E_TOT,), TRASH, jnp.int32)])
  batch_pad = jnp.concatenate(
      [batch.astype(jnp.int32), jnp.full((NPAD - N,), G, jnp.int32)])

  eye8 = jnp.eye(HEADS, dtype=f32)
  s1src = (att_src1[0][:, :, None] * eye8[:, None, :]).reshape(
      HEADS * HID, HEADS)
  s1dst = (att_dst1[0][:, :, None] * eye8[:, None, :]).reshape(
      HEADS * HID, HEADS)
  s2src = att_src2[0].reshape(OUT, 1)
  s2dst = att_dst2[0].reshape(OUT, 1)

  perm = []
  for j in range(HEADS * HID):
    base_f, w = 32 * (j // 32), j % 32
    perm.append(base_f + (2 * w if w < 16 else 2 * (w - 16) + 1))
  perm = jnp.array(perm, dtype=jnp.int32)
  bias1p = bias1[perm]
  W2p = W2[perm, :]

  z128 = jnp.zeros((NPAD, 128), f32)
  z16 = jnp.zeros((NPAD, L), f32)
  z64 = jnp.zeros((NPAD, OUT), f32)

  xp = jnp.concatenate([x, jnp.zeros((NPAD - N, F_IN), f32)], axis=0)

  def _row_blk(cols):
    return pl.BlockSpec((RB, cols), lambda i: (i, 0))

  def _full(shape):
    return pl.BlockSpec(shape, lambda i: tuple(0 for _ in shape))

  hc0, hc1, hc2, hc3, asrc16, adst16 = pl.pallas_call(
      _tc1_body,
      grid=(NPAD // RB,),
      in_specs=[_row_blk(F_IN), _full((F_IN, HEADS * HID)),
                _full((HEADS * HID, HEADS)), _full((HEADS * HID, HEADS))],
      out_specs=[_row_blk(128)] * 4 + [_row_blk(L)] * 2,
      out_shape=[jax.ShapeDtypeStruct((NPAD, 128), jnp.bfloat16)] * 4 +
                [jax.ShapeDtypeStruct((NPAD, L), f32)] * 2,
  )(xp, W1, s1src, s1dst)

  def _pack(hc):
    return lax.bitcast_convert_type(hc.reshape(NPAD, 64, 2), i32)

  hc0, hc1, hc2, hc3 = _pack(hc0), _pack(hc1), _pack(hc2), _pack(hc3)

  ex1, dpart1 = _edge_coef(asrc16, adst16, src, dst, z16)

  rd1 = pl.pallas_call(
      _recip_body,
      out_shape=jax.ShapeDtypeStruct((NPAD, L), f32),
  )(dpart1)

  o0, o1, o2, o3 = _message1(hc0, hc1, hc2, hc3, ex1, rd1, src, dst, z128)

  h2c, a2src16, a2dst16 = pl.pallas_call(
      _tc2_body,
      grid=(NPAD // RB,),
      in_specs=[_row_blk(128)] * 4 +
               [_full((HEADS * HID,)), _full((HEADS * HID, OUT)),
                _full((OUT, 1)), _full((OUT, 1))],
      out_specs=[_row_blk(OUT), _row_blk(L), _row_blk(L)],
      out_shape=[jax.ShapeDtypeStruct((NPAD, OUT), f32),
                 jax.ShapeDtypeStruct((NPAD, L), f32),
                 jax.ShapeDtypeStruct((NPAD, L), f32)],
  )(o0, o1, o2, o3, bias1p, W2p, s2src, s2dst)

  ex2, dpart2 = _edge_coef(a2src16, a2dst16, src, dst, z16)

  rd2 = pl.pallas_call(
      _recip_body,
      out_shape=jax.ShapeDtypeStruct((NPAD, L), f32),
  )(dpart2)

  (opart,) = _message2(h2c, ex2, rd2, src, dst, z64)

  pooled, h = pl.pallas_call(
      _tc3_body,
      out_shape=[jax.ShapeDtypeStruct((G, OUT), f32),
                 jax.ShapeDtypeStruct((N, OUT), f32)],
  )(opart[0], opart[1], bias2, batch_pad)

  return (pooled, h)


# speedup vs baseline: 31.3655x; 1.1883x over previous
import functools

import jax
import jax.numpy as jnp
from jax import lax
from jax.experimental import pallas as pl
from jax.experimental.pallas import tpu as pltpu
from jax.experimental.pallas import tpu_sc as plsc

N = 10000
E = 320000
F_IN = 128
HID = 64
HEADS = 8
OUT = 64
G = 64

NC = 2
NS = 16
L = 16

TRASH = N
NPAD = 10240
ROWS = NPAD // NS
RB = 1024

EB = 128
E_TOT = E + N
NBLK_ALL = 84
E_PAD = NC * NS * EB * NBLK_ALL
EPT = E_PAD // (NC * NS)
EPS = E_PAD // NS
NBLK_SC = EPS // EB
EBM = 64
NBLK_M = EPS // EBM

_MESH = plsc.VectorSubcoreMesh(
    core_axis_name="c", subcore_axis_name="s", num_cores=NC, num_subcores=NS)

f32 = jnp.float32
i32 = jnp.int32


def _dot(a, b):
  return jnp.dot(a, b, precision=lax.Precision.HIGHEST,
                 preferred_element_type=f32)


def _tc1_body(x_ref, w1_ref, ssrc_ref, sdst_ref,
              hc0, hc1, hc2, hc3, asrc16, adst16):
  h = _dot(x_ref[:], w1_ref[:])
  hb = h.astype(jnp.bfloat16)
  hc0[:] = hb[:, 0:128]
  hc1[:] = hb[:, 128:256]
  hc2[:] = hb[:, 256:384]
  hc3[:] = hb[:, 384:512]
  a_s = _dot(h, ssrc_ref[:])
  a_d = _dot(h, sdst_ref[:])
  z8 = jnp.zeros((RB, L - HEADS), f32)
  asrc16[:] = jnp.concatenate([a_s, z8], axis=1)
  adst16[:] = jnp.concatenate([a_d, z8], axis=1)


def _tc2_body(o0, o1, o2, o3, b1_ref, w2_ref, s2s_ref, s2d_ref,
              h2c, a2src16, a2dst16):
  i = pl.program_id(0)
  h1 = jnp.concatenate(
      [o0[:], o1[:], o2[:], o3[:]], axis=1) + b1_ref[:]
  h1 = jnp.where(h1 > 0, h1, jnp.exp(h1) - 1.0)
  rowid = i * RB + lax.broadcasted_iota(i32, (RB, 1), 0)
  h2 = _dot(h1, w2_ref[:])
  h2 = jnp.where(rowid < N, h2, 0.0)
  h2c[:] = h2.astype(jnp.bfloat16)
  a_s = _dot(h2, s2s_ref[:])
  a_d = _dot(h2, s2d_ref[:])
  z15 = jnp.zeros((RB, L - 1), f32)
  a2src16[:] = jnp.concatenate([a_s, z15], axis=1)
  a2dst16[:] = jnp.concatenate([a_d, z15], axis=1)


def _recip_body(dpart_ref, r_ref):
  r_ref[:] = 1.0 / (dpart_ref[0] + dpart_ref[1] + 1e-16)


def _tc3_body(p0, p1, b2_ref, batch_ref, p2m_ref, pooled_ref, h_ref):
  hs = p0[:] + p1[:]
  rowid = lax.broadcasted_iota(i32, (NPAD, 1), 0)
  hz = jnp.where(rowid < N, hs, 0.0)
  hz = _dot(hz, p2m_ref[:])
  mask = (batch_ref[:][None, :] ==
          lax.broadcasted_iota(i32, (G, NPAD), 0)).astype(f32)
  s = _dot(mask, hz)
  cnt = jnp.sum(mask, axis=1, keepdims=True)
  pooled_ref[:] = s / jnp.maximum(cnt, 1.0) + b2_ref[:]
  h_ref[:] = hz[:N, :] + b2_ref[:]


def _lane_splat(v16, lane):
  idx = jnp.full((L, 1), lane, i32)
  return lax.gather(
      v16, idx,
      lax.GatherDimensionNumbers(offset_dims=(), collapsed_slice_dims=(0,),
                                 start_index_map=(0,)),
      slice_sizes=(1,),
      mode=lax.GatherScatterMode.PROMISE_IN_BOUNDS)


def _scale_rows(hbuf, exbuf, rdbuf, i, lanes, nv):
  at16 = exbuf[i, :] * rdbuf[i, :]
  scales = [_lane_splat(at16, l) for l in lanes]
  per = nv // len(lanes)
  for v in range(nv):
    sc = scales[v // per]
    hbuf[i, pl.ds(v * L, L)] = hbuf[i, pl.ds(v * L, L)] * sc


def _scale_rows_packed(hbuf, sbuf, exbuf, rdbuf, i, lanes, ngroups=4):
  at16 = exbuf[i, :] * rdbuf[i, :]
  scales = [_lane_splat(at16, l) for l in lanes]
  per = ngroups // len(lanes)
  for g in range(ngroups):
    w = hbuf[i, pl.ds(g * L, L)]
    lo = lax.bitcast_convert_type(lax.shift_left(w, 16), f32)
    hi = lax.bitcast_convert_type(
        lax.bitwise_and(w, jnp.int32(-65536)), f32)
    sc = scales[g // per]
    sbuf[i, pl.ds(g * 2 * L, L)] = lo * sc
    sbuf[i, pl.ds((g * 2 + 1) * L, L)] = hi * sc


def _run_pipeline(nblk, prefetch, launch, finish):
  for t in range(4):
    prefetch(t, t)
  launch(0, 0, 0)

  def quad(q, carry):
    b4 = 4 * q
    for c in range(4):
      b = b4 + c
      sn, tn = (c + 1) % 2, (c + 1) % 4

      @pl.when(b + 1 < nblk)
      def _():
        launch(sn, tn, b + 1)

      finish(c % 2, c, b)
    return carry

  lax.fori_loop(0, nblk // 4, quad, 0)


@functools.partial(
    pl.kernel,
    out_type=[jax.ShapeDtypeStruct((E_PAD, L), f32),
              jax.ShapeDtypeStruct((NC, NPAD, L), f32)],
    mesh=_MESH,
    compiler_params=pltpu.CompilerParams(use_tc_tiling_on_sc=False),
    scratch_types=[
        [pltpu.VMEM((EB,), i32)] * 4,
        [pltpu.VMEM((EB,), i32)] * 4,
        [pltpu.VMEM((EB, L), f32)] * 2,
        [pltpu.VMEM((EB, L), f32)] * 2,
        [pltpu.VMEM((EB, L), f32)] * 2,
        pltpu.VMEM_SHARED((NPAD, L), f32),
        [pltpu.SemaphoreType.DMA] * 4,
        [pltpu.SemaphoreType.DMA] * 2,
        [pltpu.SemaphoreType.DMA] * 2,
    ])
def _edge_coef(asrc_hbm, adst_hbm, src_hbm, dst_hbm, z16_hbm,
               ex_hbm, dpart_hbm,
               srcv, dstv, abuf, bbuf, exbuf, dacc, semi, semg, semsc):
  cid = lax.axis_index("c")
  sid = lax.axis_index("s")
  wid = sid * NC + cid
  pltpu.sync_copy(z16_hbm.at[pl.ds(sid * ROWS, ROWS)],
                  dacc.at[pl.ds(sid * ROWS, ROWS)])
  plsc.subcore_barrier()
  ebase = wid * EPT

  def prefetch(t, b):
    off = ebase + b * EB
    pltpu.async_copy(src_hbm.at[pl.ds(off, EB)], srcv[t], semi[t])
    pltpu.async_copy(dst_hbm.at[pl.ds(off, EB)], dstv[t], semi[t])

  def launch(s, t, b):
    @pl.when(b >= 2)
    def _():
      pltpu.make_async_copy(exbuf[s], dacc.at[dstv[(t + 2) % 4]], semsc[s]).wait()

      @pl.when(b + 2 < NBLK_ALL)
      def _():
        prefetch((t + 2) % 4, b + 2)

    pltpu.make_async_copy(src_hbm.at[pl.ds(0, EB)], srcv[t], semi[t]).wait()
    pltpu.make_async_copy(dst_hbm.at[pl.ds(0, EB)], dstv[t], semi[t]).wait()
    pltpu.async_copy(asrc_hbm.at[srcv[t]], abuf[s], semg[s])
    pltpu.async_copy(adst_hbm.at[dstv[t]], bbuf[s], semg[s])

  def finish(s, t, b):
    pltpu.make_async_copy(asrc_hbm.at[srcv[t]], abuf[s], semg[s]).wait()
    pltpu.make_async_copy(adst_hbm.at[dstv[t]], bbuf[s], semg[s]).wait()

    @plsc.parallel_loop(0, EB, unroll=4)
    def body(i):
      v = abuf[s][i, :] + bbuf[s][i, :]
      v = jnp.where(v >= 0.0, v, 0.2 * v)
      exbuf[s][i, :] = jnp.exp(v)

    pltpu.sync_copy(exbuf[s], ex_hbm.at[pl.ds(ebase + b * EB, EB)])
    pltpu.async_copy(exbuf[s], dacc.at[dstv[t]], semsc[s], add=True)

  _run_pipeline(NBLK_ALL, prefetch, launch, finish)
  pltpu.make_async_copy(exbuf[0], dacc.at[dstv[2]], semsc[0]).wait()
  pltpu.make_async_copy(exbuf[1], dacc.at[dstv[3]], semsc[1]).wait()
  plsc.subcore_barrier()
  pltpu.sync_copy(dacc.at[pl.ds(sid * ROWS, ROWS)],
                  dpart_hbm.at[cid, pl.ds(sid * ROWS, ROWS)])


@functools.partial(
    pl.kernel,
    out_type=[jax.ShapeDtypeStruct((NPAD, 128), f32)] * 4,
    mesh=_MESH,
    compiler_params=pltpu.CompilerParams(use_tc_tiling_on_sc=False),
    scratch_types=[
        [pltpu.VMEM((EBM,), i32)] * 4,
        [pltpu.VMEM((EBM,), i32)] * 4,
        [pltpu.VMEM((EBM, L), f32)] * 2,
        [pltpu.VMEM((EBM, L), f32)] * 2,
        [pltpu.VMEM((EBM, 64), i32)] * 2,
        [pltpu.VMEM((EBM, 128), f32)] * 2,
        pltpu.VMEM_SHARED((NPAD, 128), f32),
        [pltpu.SemaphoreType.DMA] * 4,
        [pltpu.SemaphoreType.DMA] * 2,
        [pltpu.SemaphoreType.DMA] * 2,
    ])
def _message1(hc0, hc1, hc2, hc3, ex_hbm, rd_hbm, src_hbm, dst_hbm, z128_hbm,
              o0, o1, o2, o3,
              srcv, dstv, exbuf, rdbuf, hbuf, sbuf, acc, semi, semg, semsc):
  cid = lax.axis_index("c")
  sid = lax.axis_index("s")
  ebase = sid * EPS

  def prefetch(t, b):
    off = ebase + b * EBM
    pltpu.async_copy(src_hbm.at[pl.ds(off, EBM)], srcv[t], semi[t])
    pltpu.async_copy(dst_hbm.at[pl.ds(off, EBM)], dstv[t], semi[t])

  def do_chunk(hc, out_hbm, lanes):
    pltpu.sync_copy(z128_hbm.at[pl.ds(sid * ROWS, ROWS)],
                    acc.at[pl.ds(sid * ROWS, ROWS)])
    plsc.subcore_barrier()

    def launch(s, t, b):
      @pl.when(b >= 2)
      def _():
        pltpu.make_async_copy(sbuf[s], acc.at[dstv[(t + 2) % 4]], semsc[s]).wait()

        @pl.when(b + 2 < NBLK_M)
        def _():
          prefetch((t + 2) % 4, b + 2)

      pltpu.make_async_copy(src_hbm.at[pl.ds(0, EBM)], srcv[t], semi[t]).wait()
      pltpu.make_async_copy(dst_hbm.at[pl.ds(0, EBM)], dstv[t], semi[t]).wait()
      pltpu.async_copy(hc.at[srcv[t]], hbuf[s], semg[s])
      pltpu.async_copy(rd_hbm.at[dstv[t]], rdbuf[s], semg[s])
      pltpu.async_copy(ex_hbm.at[pl.ds(ebase + b * EBM, EBM)], exbuf[s], semg[s])

    def finish(s, t, b):
      pltpu.make_async_copy(hc.at[srcv[t]], hbuf[s], semg[s]).wait()
      pltpu.make_async_copy(rd_hbm.at[dstv[t]], rdbuf[s], semg[s]).wait()
      pltpu.make_async_copy(ex_hbm.at[pl.ds(0, EBM)], exbuf[s], semg[s]).wait()

      @plsc.parallel_loop(0, EBM, unroll=2)
      def body(i):
        _scale_rows_packed(hbuf[s], sbuf[s], exbuf[s], rdbuf[s], i, lanes)

      pltpu.async_copy(sbuf[s], acc.at[dstv[t]], semsc[s], add=True)

    _run_pipeline(NBLK_M, prefetch, launch, finish)
    pltpu.make_async_copy(sbuf[0], acc.at[dstv[2]], semsc[0]).wait()
    pltpu.make_async_copy(sbuf[1], acc.at[dstv[3]], semsc[1]).wait()
    plsc.subcore_barrier()
    pltpu.sync_copy(acc.at[pl.ds(sid * ROWS, ROWS)],
                    out_hbm.at[pl.ds(sid * ROWS, ROWS)])
    plsc.subcore_barrier()

  @pl.when(cid == 0)
  def _():
    do_chunk(hc0, o0, (0, 1))
    do_chunk(hc1, o1, (2, 3))

  @pl.when(cid == 1)
  def _():
    do_chunk(hc2, o2, (4, 5))
    do_chunk(hc3, o3, (6, 7))


@functools.partial(
    pl.kernel,
    out_type=[jax.ShapeDtypeStruct((NC, NPAD, OUT), f32)],
    mesh=_MESH,
    compiler_params=pltpu.CompilerParams(use_tc_tiling_on_sc=False),
    scratch_types=[
        [pltpu.VMEM((EB,), i32)] * 4,
        [pltpu.VMEM((EB,), i32)] * 4,
        [pltpu.VMEM((EB, L), f32)] * 2,
        [pltpu.VMEM((EB, L), f32)] * 2,
        [pltpu.VMEM((EB, OUT // 2), i32)] * 2,
        [pltpu.VMEM((EB, OUT), f32)] * 2,
        pltpu.VMEM_SHARED((NPAD, OUT), f32),
        [pltpu.SemaphoreType.DMA] * 4,
        [pltpu.SemaphoreType.DMA] * 2,
        [pltpu.SemaphoreType.DMA] * 2,
    ])
def _message2(h2c, ex_hbm, rd_hbm, src_hbm, dst_hbm, z64_hbm,
              opart,
              srcv, dstv, exbuf, rdbuf, hbuf, sbuf, acc, semi, semg, semsc):
  cid = lax.axis_index("c")
  sid = lax.axis_index("s")
  wid = sid * NC + cid
  ebase = wid * EPT
  pltpu.sync_copy(z64_hbm.at[pl.ds(sid * ROWS, ROWS)],
                  acc.at[pl.ds(sid * ROWS, ROWS)])
  plsc.subcore_barrier()

  def prefetch(t, b):
    off = ebase + b * EB
    pltpu.async_copy(src_hbm.at[pl.ds(off, EB)], srcv[t], semi[t])
    pltpu.async_copy(dst_hbm.at[pl.ds(off, EB)], dstv[t], semi[t])

  def launch(s, t, b):
    @pl.when(b >= 2)
    def _():
      pltpu.make_async_copy(sbuf[s], acc.at[dstv[(t + 2) % 4]], semsc[s]).wait()

      @pl.when(b + 2 < NBLK_ALL)
      def _():
        prefetch((t + 2) % 4, b + 2)

    pltpu.make_async_copy(src_hbm.at[pl.ds(0, EB)], srcv[t], semi[t]).wait()
    pltpu.make_async_copy(dst_hbm.at[pl.ds(0, EB)], dstv[t], semi[t]).wait()
    pltpu.async_copy(h2c.at[srcv[t]], hbuf[s], semg[s])
    pltpu.async_copy(rd_hbm.at[dstv[t]], rdbuf[s], semg[s])
    pltpu.async_copy(ex_hbm.at[pl.ds(ebase + b * EB, EB)], exbuf[s], semg[s])

  def finish(s, t, b):
    pltpu.make_async_copy(h2c.at[srcv[t]], hbuf[s], semg[s]).wait()
    pltpu.make_async_copy(rd_hbm.at[dstv[t]], rdbuf[s], semg[s]).wait()
    pltpu.make_async_copy(ex_hbm.at[pl.ds(0, EB)], exbuf[s], semg[s]).wait()

    @plsc.parallel_loop(0, EB, unroll=2)
    def body(i):
      _scale_rows_packed(hbuf[s], sbuf[s], exbuf[s], rdbuf[s], i, (0,),
                         ngroups=2)

    pltpu.async_copy(sbuf[s], acc.at[dstv[t]], semsc[s], add=True)

  _run_pipeline(NBLK_ALL, prefetch, launch, finish)
  pltpu.make_async_copy(sbuf[0], acc.at[dstv[2]], semsc[0]).wait()
  pltpu.make_async_copy(sbuf[1], acc.at[dstv[3]], semsc[1]).wait()
  plsc.subcore_barrier()
  pltpu.sync_copy(acc.at[pl.ds(sid * ROWS, ROWS)],
                  opart.at[cid, pl.ds(sid * ROWS, ROWS)])


def kernel(x, edge_index, batch, W1, att_src1, att_dst1, bias1,
           W2, att_src2, att_dst2, bias2):
  loop = jnp.arange(N, dtype=jnp.int32)
  src = jnp.concatenate(
      [edge_index[0].astype(jnp.int32), loop,
       jnp.zeros((E_PAD - E_TOT,), jnp.int32)])
  dst = jnp.concatenate(
      [edge_index[1].astype(jnp.int32), loop,
       jnp.full((E_PAD - E_TOT,), TRASH, jnp.int32)])
  batch_pad = jnp.concatenate(
      [batch.astype(jnp.int32), jnp.full((NPAD - N,), G, jnp.int32)])

  eye8 = jnp.eye(HEADS, dtype=f32)
  s1src = (att_src1[0][:, :, None] * eye8[:, None, :]).reshape(
      HEADS * HID, HEADS)
  s1dst = (att_dst1[0][:, :, None] * eye8[:, None, :]).reshape(
      HEADS * HID, HEADS)
  s2src = att_src2[0].reshape(OUT, 1)
  s2dst = att_dst2[0].reshape(OUT, 1)

  perm = []
  for j in range(HEADS * HID):
    base_f, w = 32 * (j // 32), j % 32
    perm.append(base_f + (2 * w if w < 16 else 2 * (w - 16) + 1))
  perm = jnp.array(perm, dtype=jnp.int32)
  bias1p = bias1[perm]
  W2p = W2[perm, :]
  perm2 = []
  for j in range(OUT):
    base_f, w = 32 * (j // 32), j % 32
    perm2.append(base_f + (2 * w if w < 16 else 2 * (w - 16) + 1))
  perm2 = jnp.array(perm2, dtype=jnp.int32)
  pmat = (perm2[:, None] == jnp.arange(OUT)[None, :]).astype(f32)

  z128 = jnp.zeros((NPAD, 128), f32)
  z16 = jnp.zeros((NPAD, L), f32)
  z64 = jnp.zeros((NPAD, OUT), f32)

  xp = jnp.concatenate([x, jnp.zeros((NPAD - N, F_IN), f32)], axis=0)

  def _row_blk(cols):
    return pl.BlockSpec((RB, cols), lambda i: (i, 0))

  def _full(shape):
    return pl.BlockSpec(shape, lambda i: tuple(0 for _ in shape))

  hc0, hc1, hc2, hc3, asrc16, adst16 = pl.pallas_call(
      _tc1_body,
      grid=(NPAD // RB,),
      in_specs=[_row_blk(F_IN), _full((F_IN, HEADS * HID)),
                _full((HEADS * HID, HEADS)), _full((HEADS * HID, HEADS))],
      out_specs=[_row_blk(128)] * 4 + [_row_blk(L)] * 2,
      out_shape=[jax.ShapeDtypeStruct((NPAD, 128), jnp.bfloat16)] * 4 +
                [jax.ShapeDtypeStruct((NPAD, L), f32)] * 2,
  )(xp, W1, s1src, s1dst)

  def _pack(hc):
    return lax.bitcast_convert_type(hc.reshape(NPAD, 64, 2), i32)

  hc0, hc1, hc2, hc3 = _pack(hc0), _pack(hc1), _pack(hc2), _pack(hc3)

  ex1, dpart1 = _edge_coef(asrc16, adst16, src, dst, z16)

  rd1 = pl.pallas_call(
      _recip_body,
      out_shape=jax.ShapeDtypeStruct((NPAD, L), f32),
  )(dpart1)

  o0, o1, o2, o3 = _message1(hc0, hc1, hc2, hc3, ex1, rd1, src, dst, z128)

  h2c, a2src16, a2dst16 = pl.pallas_call(
      _tc2_body,
      grid=(NPAD // RB,),
      in_specs=[_row_blk(128)] * 4 +
               [_full((HEADS * HID,)), _full((HEADS * HID, OUT)),
                _full((OUT, 1)), _full((OUT, 1))],
      out_specs=[_row_blk(OUT), _row_blk(L), _row_blk(L)],
      out_shape=[jax.ShapeDtypeStruct((NPAD, OUT), jnp.bfloat16),
                 jax.ShapeDtypeStruct((NPAD, L), f32),
                 jax.ShapeDtypeStruct((NPAD, L), f32)],
  )(o0, o1, o2, o3, bias1p, W2p, s2src, s2dst)

  ex2, dpart2 = _edge_coef(a2src16, a2dst16, src, dst, z16)

  rd2 = pl.pallas_call(
      _recip_body,
      out_shape=jax.ShapeDtypeStruct((NPAD, L), f32),
  )(dpart2)

  h2ci = lax.bitcast_convert_type(h2c.reshape(NPAD, OUT // 2, 2), i32)
  (opart,) = _message2(h2ci, ex2, rd2, src, dst, z64)

  pooled, h = pl.pallas_call(
      _tc3_body,
      out_shape=[jax.ShapeDtypeStruct((G, OUT), f32),
                 jax.ShapeDtypeStruct((N, OUT), f32)],
  )(opart[0], opart[1], bias2, batch_pad, pmat)

  return (pooled, h)


# speedup vs baseline: 31.3659x; 1.0000x over previous
import functools

import jax
import jax.numpy as jnp
from jax import lax
from jax.experimental import pallas as pl
from jax.experimental.pallas import tpu as pltpu
from jax.experimental.pallas import tpu_sc as plsc

N = 10000
E = 320000
F_IN = 128
HID = 64
HEADS = 8
OUT = 64
G = 64

NC = 2
NS = 16
L = 16

TRASH = N
NPAD = 10240
ROWS = NPAD // NS
RB = 1024

EB = 128
E_TOT = E + N
NBLK_ALL = 84
E_PAD = NC * NS * EB * NBLK_ALL
EPT = E_PAD // (NC * NS)
EPS = E_PAD // NS
NBLK_SC = EPS // EB
EBM = 64
NBLK_M = EPS // EBM

_MESH = plsc.VectorSubcoreMesh(
    core_axis_name="c", subcore_axis_name="s", num_cores=NC, num_subcores=NS)

f32 = jnp.float32
i32 = jnp.int32


def _dot(a, b):
  return jnp.dot(a, b, precision=lax.Precision.HIGHEST,
                 preferred_element_type=f32)


def _tc1_body(x_ref, w1_ref, ssrc_ref, sdst_ref,
              hc0, hc1, hc2, hc3, asrc16, adst16):
  h = _dot(x_ref[:], w1_ref[:])
  hb = h.astype(jnp.bfloat16)
  hc0[:] = hb[:, 0:128]
  hc1[:] = hb[:, 128:256]
  hc2[:] = hb[:, 256:384]
  hc3[:] = hb[:, 384:512]
  a_s = _dot(h, ssrc_ref[:])
  a_d = _dot(h, sdst_ref[:])
  z8 = jnp.zeros((RB, L - HEADS), f32)
  asrc16[:] = jnp.concatenate([a_s, z8], axis=1)
  adst16[:] = jnp.concatenate([a_d, z8], axis=1)


def _tc2_body(o0, o1, o2, o3, b1_ref, w2_ref, s2s_ref, s2d_ref,
              h2c, a2src16, a2dst16):
  i = pl.program_id(0)
  h1 = jnp.concatenate(
      [o0[:], o1[:], o2[:], o3[:]], axis=1) + b1_ref[:]
  h1 = jnp.where(h1 > 0, h1, jnp.exp(h1) - 1.0)
  rowid = i * RB + lax.broadcasted_iota(i32, (RB, 1), 0)
  h2 = _dot(h1, w2_ref[:])
  h2 = jnp.where(rowid < N, h2, 0.0)
  h2c[:] = h2.astype(jnp.bfloat16)
  a_s = _dot(h2, s2s_ref[:])
  a_d = _dot(h2, s2d_ref[:])
  z15 = jnp.zeros((RB, L - 1), f32)
  a2src16[:] = jnp.concatenate([a_s, z15], axis=1)
  a2dst16[:] = jnp.concatenate([a_d, z15], axis=1)


def _recip_body(dpart_ref, r_ref):
  r_ref[:] = 1.0 / (dpart_ref[0] + dpart_ref[1] + 1e-16)


def _tc3_body(p0, p1, b2_ref, batch_ref, p2m_ref, pooled_ref, h_ref):
  hs = p0[:] + p1[:]
  rowid = lax.broadcasted_iota(i32, (NPAD, 1), 0)
  hz = jnp.where(rowid < N, hs, 0.0)
  hz = _dot(hz, p2m_ref[:])
  mask = (batch_ref[:][None, :] ==
          lax.broadcasted_iota(i32, (G, NPAD), 0)).astype(f32)
  s = _dot(mask, hz)
  cnt = jnp.sum(mask, axis=1, keepdims=True)
  pooled_ref[:] = s / jnp.maximum(cnt, 1.0) + b2_ref[:]
  h_ref[:] = hz[:N, :] + b2_ref[:]


def _lane_splat(v16, lane):
  idx = jnp.full((L, 1), lane, i32)
  return lax.gather(
      v16, idx,
      lax.GatherDimensionNumbers(offset_dims=(), collapsed_slice_dims=(0,),
                                 start_index_map=(0,)),
      slice_sizes=(1,),
      mode=lax.GatherScatterMode.PROMISE_IN_BOUNDS)


def _scale_rows(hbuf, exbuf, rdbuf, i, lanes, nv):
  at16 = exbuf[i, :] * rdbuf[i, :]
  scales = [_lane_splat(at16, l) for l in lanes]
  per = nv // len(lanes)
  for v in range(nv):
    sc = scales[v // per]
    hbuf[i, pl.ds(v * L, L)] = hbuf[i, pl.ds(v * L, L)] * sc


def _scale_rows_packed(hbuf, sbuf, exbuf, rdbuf, i, lanes, ngroups=4):
  at16 = exbuf[i, :] * rdbuf[i, :]
  scales = [_lane_splat(at16, l) for l in lanes]
  per = ngroups // len(lanes)
  for g in range(ngroups):
    w = hbuf[i, pl.ds(g * L, L)]
    lo = lax.bitcast_convert_type(lax.shift_left(w, 16), f32)
    hi = lax.bitcast_convert_type(
        lax.bitwise_and(w, jnp.int32(-65536)), f32)
    sc = scales[g // per]
    sbuf[i, pl.ds(g * 2 * L, L)] = lo * sc
    sbuf[i, pl.ds((g * 2 + 1) * L, L)] = hi * sc


def _run_pipeline(nblk, prefetch, launch, finish):
  for t in range(4):
    prefetch(t, t)
  launch(0, 0, 0)

  def quad(q, carry):
    b4 = 4 * q
    for c in range(4):
      b = b4 + c
      sn, tn = (c + 1) % 2, (c + 1) % 4

      @pl.when(b + 1 < nblk)
      def _():
        launch(sn, tn, b + 1)

      finish(c % 2, c, b)
    return carry

  lax.fori_loop(0, nblk // 4, quad, 0)


@functools.partial(
    pl.kernel,
    out_type=[jax.ShapeDtypeStruct((E_PAD, L), f32),
              jax.ShapeDtypeStruct((NC, NPAD, L), f32)],
    mesh=_MESH,
    compiler_params=pltpu.CompilerParams(use_tc_tiling_on_sc=False),
    scratch_types=[
        [pltpu.VMEM((EB,), i32)] * 4,
        [pltpu.VMEM((EB,), i32)] * 4,
        [pltpu.VMEM((EB, L), f32)] * 2,
        [pltpu.VMEM((EB, L), f32)] * 2,
        [pltpu.VMEM((EB, L), f32)] * 2,
        pltpu.VMEM_SHARED((NPAD, L), f32),
        [pltpu.SemaphoreType.DMA] * 4,
        [pltpu.SemaphoreType.DMA] * 2,
        [pltpu.SemaphoreType.DMA] * 2,
    ])
def _edge_coef(asrc_hbm, adst_hbm, src_hbm, dst_hbm, z16_hbm,
               ex_hbm, dpart_hbm,
               srcv, dstv, abuf, bbuf, exbuf, dacc, semi, semg, semsc):
  cid = lax.axis_index("c")
  sid = lax.axis_index("s")
  wid = sid * NC + cid
  pltpu.sync_copy(z16_hbm.at[pl.ds(sid * ROWS, ROWS)],
                  dacc.at[pl.ds(sid * ROWS, ROWS)])
  plsc.subcore_barrier()
  ebase = wid * EPT

  def prefetch(t, b):
    off = ebase + b * EB
    pltpu.async_copy(src_hbm.at[pl.ds(off, EB)], srcv[t], semi[t])
    pltpu.async_copy(dst_hbm.at[pl.ds(off, EB)], dstv[t], semi[t])

  def launch(s, t, b):
    @pl.when(b >= 2)
    def _():
      pltpu.make_async_copy(exbuf[s], dacc.at[dstv[(t + 2) % 4]], semsc[s]).wait()

      @pl.when(b + 2 < NBLK_ALL)
      def _():
        prefetch((t + 2) % 4, b + 2)

    pltpu.make_async_copy(src_hbm.at[pl.ds(0, EB)], srcv[t], semi[t]).wait()
    pltpu.make_async_copy(dst_hbm.at[pl.ds(0, EB)], dstv[t], semi[t]).wait()
    pltpu.async_copy(asrc_hbm.at[srcv[t]], abuf[s], semg[s])
    pltpu.async_copy(adst_hbm.at[dstv[t]], bbuf[s], semg[s])

  def finish(s, t, b):
    pltpu.make_async_copy(asrc_hbm.at[srcv[t]], abuf[s], semg[s]).wait()
    pltpu.make_async_copy(adst_hbm.at[dstv[t]], bbuf[s], semg[s]).wait()

    @plsc.parallel_loop(0, EB, unroll=4)
    def body(i):
      v = abuf[s][i, :] + bbuf[s][i, :]
      v = jnp.where(v >= 0.0, v, 0.2 * v)
      exbuf[s][i, :] = jnp.exp(v)

    pltpu.sync_copy(exbuf[s], ex_hbm.at[pl.ds(ebase + b * EB, EB)])
    pltpu.async_copy(exbuf[s], dacc.at[dstv[t]], semsc[s], add=True)

  _run_pipeline(NBLK_ALL, prefetch, launch, finish)
  pltpu.make_async_copy(exbuf[0], dacc.at[dstv[2]], semsc[0]).wait()
  pltpu.make_async_copy(exbuf[1], dacc.at[dstv[3]], semsc[1]).wait()
  plsc.subcore_barrier()
  pltpu.sync_copy(dacc.at[pl.ds(sid * ROWS, ROWS)],
                  dpart_hbm.at[cid, pl.ds(sid * ROWS, ROWS)])


@functools.partial(
    pl.kernel,
    out_type=[jax.ShapeDtypeStruct((NPAD, 128), f32)] * 4,
    mesh=_MESH,
    compiler_params=pltpu.CompilerParams(use_tc_tiling_on_sc=False),
    scratch_types=[
        [pltpu.VMEM((EBM,), i32)] * 4,
        [pltpu.VMEM((EBM,), i32)] * 4,
        [pltpu.VMEM((EBM, L), f32)] * 2,
        [pltpu.VMEM((EBM, L), f32)] * 2,
        [pltpu.VMEM((EBM, 64), i32)] * 2,
        [pltpu.VMEM((EBM, 128), f32)] * 2,
        pltpu.VMEM_SHARED((NPAD, 128), f32),
        [pltpu.SemaphoreType.DMA] * 4,
        [pltpu.SemaphoreType.DMA] * 2,
        [pltpu.SemaphoreType.DMA] * 2,
    ])
def _message1(hc0, hc1, hc2, hc3, ex_hbm, rd_hbm, src_hbm, dst_hbm, z128_hbm,
              o0, o1, o2, o3,
              srcv, dstv, exbuf, rdbuf, hbuf, sbuf, acc, semi, semg, semsc):
  cid = lax.axis_index("c")
  sid = lax.axis_index("s")
  ebase = sid * EPS

  def prefetch(t, b):
    off = ebase + b * EBM
    pltpu.async_copy(src_hbm.at[pl.ds(off, EBM)], srcv[t], semi[t])
    pltpu.async_copy(dst_hbm.at[pl.ds(off, EBM)], dstv[t], semi[t])

  def do_chunk(hc, out_hbm, lanes):
    pltpu.sync_copy(z128_hbm.at[pl.ds(sid * ROWS, ROWS)],
                    acc.at[pl.ds(sid * ROWS, ROWS)])
    plsc.subcore_barrier()

    def launch(s, t, b):
      @pl.when(b >= 2)
      def _():
        pltpu.make_async_copy(sbuf[s], acc.at[dstv[(t + 2) % 4]], semsc[s]).wait()

        @pl.when(b + 2 < NBLK_M)
        def _():
          prefetch((t + 2) % 4, b + 2)

      pltpu.make_async_copy(src_hbm.at[pl.ds(0, EBM)], srcv[t], semi[t]).wait()
      pltpu.make_async_copy(dst_hbm.at[pl.ds(0, EBM)], dstv[t], semi[t]).wait()
      pltpu.async_copy(hc.at[srcv[t]], hbuf[s], semg[s])
      pltpu.async_copy(rd_hbm.at[dstv[t]], rdbuf[s], semg[s])
      pltpu.async_copy(ex_hbm.at[pl.ds(ebase + b * EBM, EBM)], exbuf[s], semg[s])

    def finish(s, t, b):
      pltpu.make_async_copy(hc.at[srcv[t]], hbuf[s], semg[s]).wait()
      pltpu.make_async_copy(rd_hbm.at[dstv[t]], rdbuf[s], semg[s]).wait()
      pltpu.make_async_copy(ex_hbm.at[pl.ds(0, EBM)], exbuf[s], semg[s]).wait()

      @plsc.parallel_loop(0, EBM, unroll=4)
      def body(i):
        _scale_rows_packed(hbuf[s], sbuf[s], exbuf[s], rdbuf[s], i, lanes)

      pltpu.async_copy(sbuf[s], acc.at[dstv[t]], semsc[s], add=True)

    _run_pipeline(NBLK_M, prefetch, launch, finish)
    pltpu.make_async_copy(sbuf[0], acc.at[dstv[2]], semsc[0]).wait()
    pltpu.make_async_copy(sbuf[1], acc.at[dstv[3]], semsc[1]).wait()
    plsc.subcore_barrier()
    pltpu.sync_copy(acc.at[pl.ds(sid * ROWS, ROWS)],
                    out_hbm.at[pl.ds(sid * ROWS, ROWS)])
    plsc.subcore_barrier()

  @pl.when(cid == 0)
  def _():
    do_chunk(hc0, o0, (0, 1))
    do_chunk(hc1, o1, (2, 3))

  @pl.when(cid == 1)
  def _():
    do_chunk(hc2, o2, (4, 5))
    do_chunk(hc3, o3, (6, 7))


@functools.partial(
    pl.kernel,
    out_type=[jax.ShapeDtypeStruct((NC, NPAD, OUT), f32)],
    mesh=_MESH,
    compiler_params=pltpu.CompilerParams(use_tc_tiling_on_sc=False),
    scratch_types=[
        [pltpu.VMEM((EB,), i32)] * 4,
        [pltpu.VMEM((EB,), i32)] * 4,
        [pltpu.VMEM((EB, L), f32)] * 2,
        [pltpu.VMEM((EB, L), f32)] * 2,
        [pltpu.VMEM((EB, OUT // 2), i32)] * 2,
        [pltpu.VMEM((EB, OUT), f32)] * 2,
        pltpu.VMEM_SHARED((NPAD, OUT), f32),
        [pltpu.SemaphoreType.DMA] * 4,
        [pltpu.SemaphoreType.DMA] * 2,
        [pltpu.SemaphoreType.DMA] * 2,
    ])
def _message2(h2c, ex_hbm, rd_hbm, src_hbm, dst_hbm, z64_hbm,
              opart,
              srcv, dstv, exbuf, rdbuf, hbuf, sbuf, acc, semi, semg, semsc):
  cid = lax.axis_index("c")
  sid = lax.axis_index("s")
  wid = sid * NC + cid
  ebase = wid * EPT
  pltpu.sync_copy(z64_hbm.at[pl.ds(sid * ROWS, ROWS)],
                  acc.at[pl.ds(sid * ROWS, ROWS)])
  plsc.subcore_barrier()

  def prefetch(t, b):
    off = ebase + b * EB
    pltpu.async_copy(src_hbm.at[pl.ds(off, EB)], srcv[t], semi[t])
    pltpu.async_copy(dst_hbm.at[pl.ds(off, EB)], dstv[t], semi[t])

  def launch(s, t, b):
    @pl.when(b >= 2)
    def _():
      pltpu.make_async_copy(sbuf[s], acc.at[dstv[(t + 2) % 4]], semsc[s]).wait()

      @pl.when(b + 2 < NBLK_ALL)
      def _():
        prefetch((t + 2) % 4, b + 2)

    pltpu.make_async_copy(src_hbm.at[pl.ds(0, EB)], srcv[t], semi[t]).wait()
    pltpu.make_async_copy(dst_hbm.at[pl.ds(0, EB)], dstv[t], semi[t]).wait()
    pltpu.async_copy(h2c.at[srcv[t]], hbuf[s], semg[s])
    pltpu.async_copy(rd_hbm.at[dstv[t]], rdbuf[s], semg[s])
    pltpu.async_copy(ex_hbm.at[pl.ds(ebase + b * EB, EB)], exbuf[s], semg[s])

  def finish(s, t, b):
    pltpu.make_async_copy(h2c.at[srcv[t]], hbuf[s], semg[s]).wait()
    pltpu.make_async_copy(rd_hbm.at[dstv[t]], rdbuf[s], semg[s]).wait()
    pltpu.make_async_copy(ex_hbm.at[pl.ds(0, EB)], exbuf[s], semg[s]).wait()

    @plsc.parallel_loop(0, EB, unroll=2)
    def body(i):
      _scale_rows_packed(hbuf[s], sbuf[s], exbuf[s], rdbuf[s], i, (0,),
                         ngroups=2)

    pltpu.async_copy(sbuf[s], acc.at[dstv[t]], semsc[s], add=True)

  _run_pipeline(NBLK_ALL, prefetch, launch, finish)
  pltpu.make_async_copy(sbuf[0], acc.at[dstv[2]], semsc[0]).wait()
  pltpu.make_async_copy(sbuf[1], acc.at[dstv[3]], semsc[1]).wait()
  plsc.subcore_barrier()
  pltpu.sync_copy(acc.at[pl.ds(sid * ROWS, ROWS)],
                  opart.at[cid, pl.ds(sid * ROWS, ROWS)])


def kernel(x, edge_index, batch, W1, att_src1, att_dst1, bias1,
           W2, att_src2, att_dst2, bias2):
  loop = jnp.arange(N, dtype=jnp.int32)
  src = jnp.concatenate(
      [edge_index[0].astype(jnp.int32), loop,
       jnp.zeros((E_PAD - E_TOT,), jnp.int32)])
  dst = jnp.concatenate(
      [edge_index[1].astype(jnp.int32), loop,
       jnp.full((E_PAD - E_TOT,), TRASH, jnp.int32)])
  batch_pad = jnp.concatenate(
      [batch.astype(jnp.int32), jnp.full((NPAD - N,), G, jnp.int32)])

  eye8 = jnp.eye(HEADS, dtype=f32)
  s1src = (att_src1[0][:, :, None] * eye8[:, None, :]).reshape(
      HEADS * HID, HEADS)
  s1dst = (att_dst1[0][:, :, None] * eye8[:, None, :]).reshape(
      HEADS * HID, HEADS)
  s2src = att_src2[0].reshape(OUT, 1)
  s2dst = att_dst2[0].reshape(OUT, 1)

  perm = []
  for j in range(HEADS * HID):
    base_f, w = 32 * (j // 32), j % 32
    perm.append(base_f + (2 * w if w < 16 else 2 * (w - 16) + 1))
  perm = jnp.array(perm, dtype=jnp.int32)
  bias1p = bias1[perm]
  W2p = W2[perm, :]
  perm2 = []
  for j in range(OUT):
    base_f, w = 32 * (j // 32), j % 32
    perm2.append(base_f + (2 * w if w < 16 else 2 * (w - 16) + 1))
  perm2 = jnp.array(perm2, dtype=jnp.int32)
  pmat = (perm2[:, None] == jnp.arange(OUT)[None, :]).astype(f32)

  z128 = jnp.zeros((NPAD, 128), f32)
  z16 = jnp.zeros((NPAD, L), f32)
  z64 = jnp.zeros((NPAD, OUT), f32)

  xp = jnp.concatenate([x, jnp.zeros((NPAD - N, F_IN), f32)], axis=0)

  def _row_blk(cols):
    return pl.BlockSpec((RB, cols), lambda i: (i, 0))

  def _full(shape):
    return pl.BlockSpec(shape, lambda i: tuple(0 for _ in shape))

  hc0, hc1, hc2, hc3, asrc16, adst16 = pl.pallas_call(
      _tc1_body,
      grid=(NPAD // RB,),
      in_specs=[_row_blk(F_IN), _full((F_IN, HEADS * HID)),
                _full((HEADS * HID, HEADS)), _full((HEADS * HID, HEADS))],
      out_specs=[_row_blk(128)] * 4 + [_row_blk(L)] * 2,
      out_shape=[jax.ShapeDtypeStruct((NPAD, 128), jnp.bfloat16)] * 4 +
                [jax.ShapeDtypeStruct((NPAD, L), f32)] * 2,
  )(xp, W1, s1src, s1dst)

  def _pack(hc):
    return lax.bitcast_convert_type(hc.reshape(NPAD, 64, 2), i32)

  hc0, hc1, hc2, hc3 = _pack(hc0), _pack(hc1), _pack(hc2), _pack(hc3)

  ex1, dpart1 = _edge_coef(asrc16, adst16, src, dst, z16)

  rd1 = pl.pallas_call(
      _recip_body,
      out_shape=jax.ShapeDtypeStruct((NPAD, L), f32),
  )(dpart1)

  o0, o1, o2, o3 = _message1(hc0, hc1, hc2, hc3, ex1, rd1, src, dst, z128)

  h2c, a2src16, a2dst16 = pl.pallas_call(
      _tc2_body,
      grid=(NPAD // RB,),
      in_specs=[_row_blk(128)] * 4 +
               [_full((HEADS * HID,)), _full((HEADS * HID, OUT)),
                _full((OUT, 1)), _full((OUT, 1))],
      out_specs=[_row_blk(OUT), _row_blk(L), _row_blk(L)],
      out_shape=[jax.ShapeDtypeStruct((NPAD, OUT), jnp.bfloat16),
                 jax.ShapeDtypeStruct((NPAD, L), f32),
                 jax.ShapeDtypeStruct((NPAD, L), f32)],
  )(o0, o1, o2, o3, bias1p, W2p, s2src, s2dst)

  ex2, dpart2 = _edge_coef(a2src16, a2dst16, src, dst, z16)

  rd2 = pl.pallas_call(
      _recip_body,
      out_shape=jax.ShapeDtypeStruct((NPAD, L), f32),
  )(dpart2)

  h2ci = lax.bitcast_convert_type(h2c.reshape(NPAD, OUT // 2, 2), i32)
  (opart,) = _message2(h2ci, ex2, rd2, src, dst, z64)

  pooled, h = pl.pallas_call(
      _tc3_body,
      out_shape=[jax.ShapeDtypeStruct((G, OUT), f32),
                 jax.ShapeDtypeStruct((N, OUT), f32)],
  )(opart[0], opart[1], bias2, batch_pad, pmat)

  return (pooled, h)
